# Initial kernel scaffold; baseline (speedup 1.0000x reference)
import jax
import jax.numpy as jnp
from jax.experimental import pallas as pl


def kernel(pos_seq, W0, W1):
    raise NotImplementedError("write your pallas kernel here")


# speedup vs baseline: 3.5696x; 3.5696x over previous
import functools

import jax
import jax.numpy as jnp
from jax import lax
from jax.experimental import pallas as pl
from jax.experimental.pallas import tpu as pltpu
from jax.experimental.pallas import tpu_sc as plsc

_NUM_CORES = 2
_NUM_SUBCORES = 16
_NUM_WORKERS = _NUM_CORES * _NUM_SUBCORES
_LANES = 16

_B = 4 * 8192
_PER_W = _B // _NUM_WORKERS
_CHUNK = 128
_NCHUNK = _PER_W // _CHUNK
_D = 32


def _sc_body(pos_hbm, w0_hbm, w1_hbm, out_hbm,
             pos_v, idx0_v, idx1_v, rows0_v, rows1_v, sem0, sem1):
    wid = lax.axis_index("s") * _NUM_CORES + lax.axis_index("c")
    base = wid * _PER_W

    pltpu.sync_copy(pos_hbm.at[pl.ds(base, _PER_W)], pos_v)

    @pl.loop(0, _NCHUNK)
    def _(c):
        @pl.loop(0, _CHUNK, step=_LANES)
        def _(j):
            p = pos_v[pl.ds(c * _CHUNK + j, _LANES)]
            idx0_v[c, pl.ds(j, _LANES)] = lax.bitwise_and(p, 255)
            idx1_v[c, pl.ds(j, _LANES)] = lax.bitwise_and(
                lax.shift_right_logical(p, 8), 127)

    @pl.loop(0, _NCHUNK)
    def _(c):
        off = c * _CHUNK
        g0 = pltpu.async_copy(
            w0_hbm.at[idx0_v.at[c]], rows0_v.at[pl.ds(off, _CHUNK), :], sem0)
        g1 = pltpu.async_copy(
            w1_hbm.at[idx1_v.at[c]], rows1_v.at[pl.ds(off, _CHUNK), :], sem1)
        g0.wait()
        g1.wait()

    pltpu.sync_copy(rows0_v, out_hbm.at[pl.ds(base, _PER_W), pl.ds(0, _D)])
    pltpu.sync_copy(rows1_v, out_hbm.at[pl.ds(base, _PER_W), pl.ds(_D, _D)])


@jax.jit
def _sc_embed(pos_flat, w0, w1):
    mesh = plsc.VectorSubcoreMesh(core_axis_name="c", subcore_axis_name="s")
    k = pl.kernel(
        _sc_body,
        mesh=mesh,
        compiler_params=pltpu.CompilerParams(use_tc_tiling_on_sc=False),
        out_type=jax.ShapeDtypeStruct((_B, 2 * _D), jnp.float32),
        scratch_types=[
            pltpu.VMEM((_PER_W,), jnp.int32),
            pltpu.VMEM((_NCHUNK, _CHUNK), jnp.int32),
            pltpu.VMEM((_NCHUNK, _CHUNK), jnp.int32),
            pltpu.VMEM((_PER_W, _D), jnp.float32),
            pltpu.VMEM((_PER_W, _D), jnp.float32),
            pltpu.SemaphoreType.DMA,
            pltpu.SemaphoreType.DMA,
        ],
    )
    return k(pos_flat, w0, w1)


def kernel(pos_seq, W0, W1):
    pos_flat = pos_seq.reshape(-1).astype(jnp.int32)
    out = _sc_embed(pos_flat, W0, W1)
    return out.reshape(*pos_seq.shape, 2 * _D)


# speedup vs baseline: 3.6551x; 1.0240x over previous
import functools

import jax
import jax.numpy as jnp
from jax import lax
from jax.experimental import pallas as pl
from jax.experimental.pallas import tpu as pltpu
from jax.experimental.pallas import tpu_sc as plsc

_NUM_CORES = 2
_NUM_SUBCORES = 16
_NUM_WORKERS = _NUM_CORES * _NUM_SUBCORES
_LANES = 16

_B = 4 * 8192
_PER_W = _B // _NUM_WORKERS
_CHUNK = 128
_NCHUNK = _PER_W // _CHUNK
_D = 32


def _sc_body(pos_hbm, w0_hbm, w1_hbm, out_hbm,
             pos_v, idx0_v, idx1_v, rows0_v, rows1_v, sem0, sem1):
    wid = lax.axis_index("s") * _NUM_CORES + lax.axis_index("c")
    base = wid * _PER_W

    pltpu.sync_copy(pos_hbm.at[pl.ds(base, _PER_W)], pos_v)

    @pl.loop(0, _NCHUNK)
    def _(c):
        @pl.loop(0, _CHUNK, step=_LANES)
        def _(j):
            p = pos_v[pl.ds(c * _CHUNK + j, _LANES)]
            idx0_v[c, pl.ds(j, _LANES)] = lax.bitwise_and(p, 255)
            idx1_v[c, pl.ds(j, _LANES)] = lax.bitwise_and(
                lax.shift_right_logical(p, 8), 127)
        off = c * _CHUNK
        pltpu.async_copy(
            w0_hbm.at[idx0_v.at[c]], rows0_v.at[pl.ds(off, _CHUNK), :], sem0)
        pltpu.async_copy(
            w1_hbm.at[idx1_v.at[c]], rows1_v.at[pl.ds(off, _CHUNK), :], sem1)

    out0 = out_hbm.at[pl.ds(base, _PER_W), pl.ds(0, _D)]
    out1 = out_hbm.at[pl.ds(base, _PER_W), pl.ds(_D, _D)]

    pltpu.make_async_copy(out0, rows0_v, sem0).wait()
    pltpu.make_async_copy(out1, rows1_v, sem1).wait()

    w0c = pltpu.async_copy(rows0_v, out0, sem0)
    w1c = pltpu.async_copy(rows1_v, out1, sem1)
    w0c.wait()
    w1c.wait()


@jax.jit
def _sc_embed(pos_flat, w0, w1):
    mesh = plsc.VectorSubcoreMesh(core_axis_name="c", subcore_axis_name="s")
    k = pl.kernel(
        _sc_body,
        mesh=mesh,
        compiler_params=pltpu.CompilerParams(use_tc_tiling_on_sc=False),
        out_type=jax.ShapeDtypeStruct((_B, 2 * _D), jnp.float32),
        scratch_types=[
            pltpu.VMEM((_PER_W,), jnp.int32),
            pltpu.VMEM((_NCHUNK, _CHUNK), jnp.int32),
            pltpu.VMEM((_NCHUNK, _CHUNK), jnp.int32),
            pltpu.VMEM((_PER_W, _D), jnp.float32),
            pltpu.VMEM((_PER_W, _D), jnp.float32),
            pltpu.SemaphoreType.DMA,
            pltpu.SemaphoreType.DMA,
        ],
    )
    return k(pos_flat, w0, w1)


def kernel(pos_seq, W0, W1):
    pos_flat = pos_seq.reshape(-1).astype(jnp.int32)
    out = _sc_embed(pos_flat, W0, W1)
    return out.reshape(*pos_seq.shape, 2 * _D)
